# Initial kernel scaffold; baseline (speedup 1.0000x reference)
import jax
import jax.numpy as jnp
from jax.experimental import pallas as pl


def kernel(input_ids, token_embedding, position_embedding, special_token_embedding):
    raise NotImplementedError("write your pallas kernel here")


# speedup vs baseline: 2.2348x; 2.2348x over previous
import functools

import jax
import jax.numpy as jnp
from jax import lax
from jax.experimental import pallas as pl
from jax.experimental.pallas import tpu as pltpu
from jax.experimental.pallas import tpu_sc as plsc

SEQ = 8192
D = 768
LANES = 16
DL = D // LANES
NC = 2
NS = 16
NW = NC * NS
ROWS_PER_W = SEQ // NW
R = 64
NCHUNK = ROWS_PER_W // R


def _sc_embed(ids, tok_table, pos_table, special):
    mesh = plsc.VectorSubcoreMesh(core_axis_name="c", subcore_axis_name="s")

    @functools.partial(
        pl.kernel,
        mesh=mesh,
        out_type=jax.ShapeDtypeStruct((SEQ, D), jnp.float32),
        scratch_types=[
            pltpu.VMEM((R,), jnp.int32),
            pltpu.VMEM((R, D), jnp.float32),
            pltpu.VMEM((R + 8, D), jnp.float32),
            pltpu.SemaphoreType.DMA,
        ],
    )
    def k(ids_hbm, tok_hbm, pos_hbm, sp_hbm, out_hbm, idx_v, tokb, posb, sem):
        wid = lax.axis_index("s") * NC + lax.axis_index("c")
        base = wid * ROWS_PER_W

        def chunk(c, carry):
            r0 = base + c * R
            pltpu.sync_copy(ids_hbm.at[pl.ds(r0, R)], idx_v)
            gather = pltpu.async_copy(tok_hbm.at[idx_v], tokb, sem)

            @pl.when(r0 == 0)
            def _():
                pltpu.sync_copy(pos_hbm.at[pl.ds(0, R)], posb.at[pl.ds(8, R)])

            @pl.when(r0 != 0)
            def _():
                pltpu.sync_copy(pos_hbm.at[pl.ds(r0 - 8, R + 8)], posb)

            gather.wait()

            def row(i, c2):
                for j in range(DL):
                    sl = pl.ds(j * LANES, LANES)
                    tokb[i, sl] = tokb[i, sl] + posb[i + 7, sl]
                return c2

            lax.fori_loop(0, R, row, 0)
            pltpu.sync_copy(tokb, out_hbm.at[pl.ds(r0, R)])
            return carry

        lax.fori_loop(0, NCHUNK, chunk, 0)

        @pl.when(wid == 0)
        def _():
            pltpu.sync_copy(sp_hbm, posb.at[pl.ds(0, 1)])
            pltpu.sync_copy(posb.at[pl.ds(0, 1)], out_hbm.at[pl.ds(0, 1)])

    return k(ids, tok_table, pos_table, special)


@jax.jit
def kernel(input_ids, token_embedding, position_embedding, special_token_embedding):
    ids = input_ids.reshape(SEQ).astype(jnp.int32)
    sp = special_token_embedding.reshape(1, D)
    out = _sc_embed(ids, token_embedding, position_embedding, sp)
    return out.reshape(1, SEQ, D)
